# baseline (speedup 1.0000x reference)
import functools

import jax
import jax.numpy as jnp
import numpy as np
from jax.experimental import pallas as pl

N_LEVELS = 16
N_FEATS = 2
LOG2_T = 19
T = 1 << LOG2_T
BASE_RES = 16
PER_LEVEL_SCALE = 1.5
PRIMES = (1, 2654435761, 805459861)
WIDTH = 64

BN = 4000


def _hash_encode_xla(xyz, table):
    n = xyz.shape[0]
    outs = []
    for l in range(N_LEVELS):
        scale = float(np.exp2(l * np.log2(PER_LEVEL_SCALE)) * BASE_RES - 1.0)
        res = int(np.ceil(scale)) + 1
        pos = xyz * scale + 0.5
        pos_f = jnp.floor(pos)
        frac = pos - pos_f
        base = pos_f.astype(jnp.int32)
        feat = jnp.zeros((n, N_FEATS), dtype=xyz.dtype)
        for corner in range(8):
            off = np.array([(corner >> d) & 1 for d in range(3)], dtype=np.int32)
            offj = jnp.asarray(off)[None, :]
            c = base + offj
            w = jnp.prod(jnp.where(offj == 1, frac, 1.0 - frac), axis=1)
            if res ** 3 <= T:
                cc = jnp.clip(c, 0, res - 1)
                idx = cc[:, 0] + cc[:, 1] * res + cc[:, 2] * res * res
            else:
                cu = c.astype(jnp.uint32)
                h = (cu[:, 0] * jnp.uint32(PRIMES[0])) ^ (cu[:, 1] * jnp.uint32(PRIMES[1])) ^ (cu[:, 2] * jnp.uint32(PRIMES[2]))
                idx = (h % jnp.uint32(T)).astype(jnp.int32)
            feat = feat + w[:, None] * jnp.take(table[l], idx, axis=0)
        outs.append(feat)
    return jnp.concatenate(outs, axis=1)


def _mlp_body(shs_ref, aux_ref, emb_ref, w1a_ref, w1b_ref, w1c_ref, w2_ref, w3_ref, out_ref):
    s = shs_ref[...]
    ss = jnp.sum(s * s, axis=1, keepdims=True)
    sn = s / jnp.maximum(jnp.sqrt(ss), 1e-12)
    h = (jnp.dot(sn, w1a_ref[...], preferred_element_type=jnp.float32)
         + jnp.dot(aux_ref[...], w1b_ref[...], preferred_element_type=jnp.float32)
         + jnp.dot(emb_ref[...], w1c_ref[...], preferred_element_type=jnp.float32))
    h = jnp.maximum(h, 0.0)
    h = jnp.maximum(jnp.dot(h, w2_ref[...], preferred_element_type=jnp.float32), 0.0)
    out_ref[...] = jnp.dot(h, w3_ref[...], preferred_element_type=jnp.float32)


def _mlp(shs, aux, emb, W1a, W1b, W1c, W2, W3):
    n = shs.shape[0]
    grid = n // BN
    full = lambda shp: pl.BlockSpec(shp, lambda i: (0, 0))
    return pl.pallas_call(
        _mlp_body,
        grid=(grid,),
        in_specs=[
            pl.BlockSpec((BN, 48), lambda i: (i, 0)),
            pl.BlockSpec((BN, 10), lambda i: (i, 0)),
            pl.BlockSpec((BN, N_LEVELS * N_FEATS), lambda i: (i, 0)),
            full((48, WIDTH)), full((10, WIDTH)), full((N_LEVELS * N_FEATS, WIDTH)),
            full((WIDTH, WIDTH)), full((WIDTH, 1)),
        ],
        out_specs=pl.BlockSpec((BN, 1), lambda i: (i, 0)),
        out_shape=jax.ShapeDtypeStruct((n, 1), jnp.float32),
    )(shs, aux, emb, W1a, W1b, W1c, W2, W3)


def kernel(shs, rotations, scales, viewdirs, xyz, table, W1, W2, W3):
    n = shs.shape[0]
    shs2 = shs.reshape(n, -1)
    aux = jnp.concatenate([viewdirs, rotations, scales], axis=1)
    emb = _hash_encode_xla(xyz, table)
    W1a, W1b, W1c = W1[:48], W1[48:58], W1[58:]
    return _mlp(shs2, aux, emb, W1a, W1b, W1c, W2, W3)


# speedup vs baseline: 2.4927x; 2.4927x over previous
import functools

import jax
import jax.numpy as jnp
import numpy as np
from jax import lax
from jax.experimental import pallas as pl
from jax.experimental.pallas import tpu as pltpu
from jax.experimental.pallas import tpu_sc as plsc

N_LEVELS = 16
N_FEATS = 2
LOG2_T = 19
T = 1 << LOG2_T
MASK = T - 1
BASE_RES = 16
PER_LEVEL_SCALE = 1.5
P2I = np.uint32(2654435761).astype(np.int32)
P3I = np.uint32(805459861).astype(np.int32)
WIDTH = 64

NW = 32
C = 512
GROUPS = C // 16
CHUNKS = 62
PW = C * CHUNKS
NP = NW * PW

_LEVELS = []
for _l in range(N_LEVELS):
    _scale = float(np.exp2(_l * np.log2(PER_LEVEL_SCALE)) * BASE_RES - 1.0)
    _res = int(np.ceil(_scale)) + 1
    _LEVELS.append((_scale, _res, _res ** 3 <= T))

BN = 4000


def _sc_body(x_hbm, y_hbm, z_hbm, tab0_hbm, tab1_hbm, emb_hbm,
             xv, yv, zv, iv0, iv1, wv0, wv1,
             r0a, r1a, r0b, r1b, ev, sem0, sem1):
    wid = lax.axis_index("s") * 2 + lax.axis_index("c")
    iota = lax.iota(jnp.int32, 16)

    def pass_a(l, ib, wb):
        scale, res, direct = _LEVELS[l]

        def body(g, _):
            x = xv[pl.ds(g * 16, 16)]
            y = yv[pl.ds(g * 16, 16)]
            z = zv[pl.ds(g * 16, 16)]
            px = x * scale + 0.5
            py = y * scale + 0.5
            pz = z * scale + 0.5
            bx = px.astype(jnp.int32)
            by = py.astype(jnp.int32)
            bz = pz.astype(jnp.int32)
            fx = px - bx.astype(jnp.float32)
            fy = py - by.astype(jnp.float32)
            fz = pz - bz.astype(jnp.float32)
            if direct:
                r2 = res * res
                off = l * T
                hx = (bx, jnp.minimum(bx + 1, res - 1))
                sy0 = by * res
                hy = (sy0, jnp.minimum(sy0 + res, (res - 1) * res))
                sz0 = bz * r2
                sz1 = jnp.minimum(sz0 + r2, (res - 1) * r2)
                hz = (sz0 + off, sz1 + off)
                comb = lambda a, b, c: a + b + c
            else:
                off = l * T
                hx = (bx, bx + 1)
                hy0 = by * P2I
                hy = (hy0, hy0 + P2I)
                hz0 = bz * P3I
                hz = (hz0, hz0 + P3I)
                comb = lambda a, b, c: ((a ^ b ^ c) & MASK) + off
            wx = (1.0 - fx, fx)
            wy = (1.0 - fy, fy)
            wz = (1.0 - fz, fz)
            wyz = [[wy[j] * wz[k] for k in range(2)] for j in range(2)]
            for c in range(8):
                cx, cy, cz = c & 1, (c >> 1) & 1, (c >> 2) & 1
                idx = comb(hx[cx], hy[cy], hz[cz])
                w = wx[cx] * wyz[cy][cz]
                p0 = g * 128 + c * 16
                ib[pl.ds(p0, 16)] = idx
                wb[pl.ds(p0, 16)] = w
            return 0

        lax.fori_loop(0, GROUPS, body, 0)

    def pass_b(l, wb, rb0, rb1):
        def body(g, _):
            a0 = jnp.zeros((16,), jnp.float32)
            a1 = jnp.zeros((16,), jnp.float32)
            for c in range(8):
                p0 = g * 128 + c * 16
                wc = wb[pl.ds(p0, 16)]
                f0 = rb0[pl.ds(p0, 16)]
                f1 = rb1[pl.ds(p0, 16)]
                a0 = a0 + wc * f0
                a1 = a1 + wc * f1
            ep = iota * 32 + (g * 512 + 2 * l)
            plsc.store_scatter(ev, [ep], a0)
            plsc.store_scatter(ev, [ep + 1], a1)
            return 0

        lax.fori_loop(0, GROUPS, body, 0)

    def chunk(k, _):
        base = wid * PW + k * C
        pltpu.sync_copy(x_hbm.at[pl.ds(base, C)], xv)
        pltpu.sync_copy(y_hbm.at[pl.ds(base, C)], yv)
        pltpu.sync_copy(z_hbm.at[pl.ds(base, C)], zv)
        cps = [None] * N_LEVELS
        for l in range(N_LEVELS):
            par = l % 2
            ib, wb = (iv0, wv0) if par == 0 else (iv1, wv1)
            rb0, rb1, sb = (r0a, r1a, sem0) if par == 0 else (r0b, r1b, sem1)
            pass_a(l, ib, wb)
            c0 = pltpu.async_copy(tab0_hbm.at[ib], rb0, sb)
            c1 = pltpu.async_copy(tab1_hbm.at[ib], rb1, sb)
            cps[l] = (c0, c1)
            if l >= 1:
                cps[l - 1][0].wait()
                cps[l - 1][1].wait()
                pp = (l - 1) % 2
                wbp = wv0 if pp == 0 else wv1
                rb0p, rb1p = (r0a, r1a) if pp == 0 else (r0b, r1b)
                pass_b(l - 1, wbp, rb0p, rb1p)
        cps[N_LEVELS - 1][0].wait()
        cps[N_LEVELS - 1][1].wait()
        pass_b(N_LEVELS - 1, wv1, r0b, r1b)
        pltpu.sync_copy(ev, emb_hbm.at[pl.ds(base * 32, C * 32)])
        return 0

    lax.fori_loop(0, CHUNKS, chunk, 0)


@functools.partial(
    pl.kernel,
    mesh=plsc.VectorSubcoreMesh(core_axis_name="c", subcore_axis_name="s"),
    compiler_params=pltpu.CompilerParams(needs_layout_passes=False),
    out_type=jax.ShapeDtypeStruct((NP * 32,), jnp.float32),
    scratch_types=[
        pltpu.VMEM((C,), jnp.float32),
        pltpu.VMEM((C,), jnp.float32),
        pltpu.VMEM((C,), jnp.float32),
        pltpu.VMEM((C * 8,), jnp.int32),
        pltpu.VMEM((C * 8,), jnp.int32),
        pltpu.VMEM((C * 8,), jnp.float32),
        pltpu.VMEM((C * 8,), jnp.float32),
        pltpu.VMEM((C * 8,), jnp.float32),
        pltpu.VMEM((C * 8,), jnp.float32),
        pltpu.VMEM((C * 8,), jnp.float32),
        pltpu.VMEM((C * 8,), jnp.float32),
        pltpu.VMEM((C * 32,), jnp.float32),
        pltpu.SemaphoreType.DMA,
        pltpu.SemaphoreType.DMA,
    ],
)
def _sc_encode(x_hbm, y_hbm, z_hbm, tab0_hbm, tab1_hbm, emb_hbm, *rest):
    _sc_body(x_hbm, y_hbm, z_hbm, tab0_hbm, tab1_hbm, emb_hbm, *rest)


def _mlp_body(shs_ref, aux_ref, emb_ref, w1a_ref, w1b_ref, w1c_ref, w2_ref, w3_ref, out_ref):
    s = shs_ref[...]
    ss = jnp.sum(s * s, axis=1, keepdims=True)
    sn = s / jnp.maximum(jnp.sqrt(ss), 1e-12)
    h = (jnp.dot(sn, w1a_ref[...], preferred_element_type=jnp.float32)
         + jnp.dot(aux_ref[...], w1b_ref[...], preferred_element_type=jnp.float32)
         + jnp.dot(emb_ref[...], w1c_ref[...], preferred_element_type=jnp.float32))
    h = jnp.maximum(h, 0.0)
    h = jnp.maximum(jnp.dot(h, w2_ref[...], preferred_element_type=jnp.float32), 0.0)
    out_ref[...] = jnp.dot(h, w3_ref[...], preferred_element_type=jnp.float32)


def _mlp(shs, aux, emb, W1a, W1b, W1c, W2, W3):
    n = shs.shape[0]
    grid = n // BN
    full = lambda shp: pl.BlockSpec(shp, lambda i: (0, 0))
    return pl.pallas_call(
        _mlp_body,
        grid=(grid,),
        in_specs=[
            pl.BlockSpec((BN, 48), lambda i: (i, 0)),
            pl.BlockSpec((BN, 10), lambda i: (i, 0)),
            pl.BlockSpec((BN, N_LEVELS * N_FEATS), lambda i: (i, 0)),
            full((48, WIDTH)), full((10, WIDTH)), full((N_LEVELS * N_FEATS, WIDTH)),
            full((WIDTH, WIDTH)), full((WIDTH, 1)),
        ],
        out_specs=pl.BlockSpec((BN, 1), lambda i: (i, 0)),
        out_shape=jax.ShapeDtypeStruct((n, 1), jnp.float32),
    )(shs, aux, emb, W1a, W1b, W1c, W2, W3)


def kernel(shs, rotations, scales, viewdirs, xyz, table, W1, W2, W3):
    n = shs.shape[0]
    shs2 = shs.reshape(n, -1)
    aux = jnp.concatenate([viewdirs, rotations, scales], axis=1)
    pad = NP - n
    xp = jnp.pad(xyz[:, 0], (0, pad))
    yp = jnp.pad(xyz[:, 1], (0, pad))
    zp = jnp.pad(xyz[:, 2], (0, pad))
    tab = table.reshape(N_LEVELS * T, N_FEATS)
    tab0 = tab[:, 0]
    tab1 = tab[:, 1]
    emb = _sc_encode(xp, yp, zp, tab0, tab1).reshape(NP, 32)[:n]
    W1a, W1b, W1c = W1[:48], W1[48:58], W1[58:]
    return _mlp(shs2, aux, emb, W1a, W1b, W1c, W2, W3)


# speedup vs baseline: 2.4928x; 1.0000x over previous
import functools

import jax
import jax.numpy as jnp
import numpy as np
from jax import lax
from jax.experimental import pallas as pl
from jax.experimental.pallas import tpu as pltpu
from jax.experimental.pallas import tpu_sc as plsc

N_LEVELS = 16
N_FEATS = 2
LOG2_T = 19
T = 1 << LOG2_T
MASK = T - 1
BASE_RES = 16
PER_LEVEL_SCALE = 1.5
P2I = np.uint32(2654435761).astype(np.int32)
P3I = np.uint32(805459861).astype(np.int32)
WIDTH = 64

NW = 32
C = 512
GROUPS = C // 16
CHUNKS = 62
PW = C * CHUNKS
NP = NW * PW

_LEVELS = []
for _l in range(N_LEVELS):
    _scale = float(np.exp2(_l * np.log2(PER_LEVEL_SCALE)) * BASE_RES - 1.0)
    _res = int(np.ceil(_scale)) + 1
    _LEVELS.append((_scale, _res, _res ** 3 <= T))

N_SP = 0
_SLAB = 4096
_SP_USED = [min(_LEVELS[l][1] ** 3, T) for l in range(N_SP)]
_SP_SIZE = [-(-u // _SLAB) * _SLAB for u in _SP_USED]
_SP_OFF = [sum(_SP_SIZE[:l]) for l in range(N_SP)]
SP_ROWS = max(sum(_SP_SIZE), 8)
_FILL = [(l * T + s * _SLAB, _SP_OFF[l] + s * _SLAB)
         for l in range(N_SP) for s in range(_SP_SIZE[l] // _SLAB)]

BN = 4000


def _sc_body(x_hbm, y_hbm, z_hbm, tab0_hbm, tab1_hbm, emb_hbm,
             xv, yv, zv, iv0, iv1, wv0, wv1,
             rpa, rpb, r0a, r1a, r0b, r1b, ev, st0, st1, v2, spt, sem0, sem1):
    sid = lax.axis_index("s")
    wid = sid * 2 + lax.axis_index("c")
    iota = lax.iota(jnp.int32, 16)
    zidx = jnp.zeros((16,), jnp.int32)
    oidx = jnp.full((16,), 1, jnp.int32)

    prefix = 0
    for l in range(N_SP):
        nsl = _SP_SIZE[l] // _SLAB

        def fil(s, _, l=l, prefix=prefix):
            @pl.when(((s + prefix) % 16) == sid)
            def _():
                src0 = l * T + s * _SLAB
                dst0 = _SP_OFF[l] + s * _SLAB
                pltpu.sync_copy(tab0_hbm.at[pl.ds(src0, _SLAB)], st0)
                pltpu.sync_copy(tab1_hbm.at[pl.ds(src0, _SLAB)], st1)

                def rep(j, _):
                    x0 = st0[pl.ds(j * 16, 16)]
                    x1 = st1[pl.ds(j * 16, 16)]
                    plsc.store_scatter(v2, [j * 16 + iota, zidx], x0)
                    plsc.store_scatter(v2, [j * 16 + iota, oidx], x1)
                    return 0

                lax.fori_loop(0, _SLAB // 16, rep, 0)
                pltpu.sync_copy(v2, spt.at[pl.ds(dst0, _SLAB), :])

            return 0

        lax.fori_loop(0, nsl, fil, 0)
        prefix += nsl

    plsc.subcore_barrier()

    def pass_a(l, ib, wb):
        scale, res, direct = _LEVELS[l]

        def body(g, _):
            x = xv[pl.ds(g * 16, 16)]
            y = yv[pl.ds(g * 16, 16)]
            z = zv[pl.ds(g * 16, 16)]
            px = x * scale + 0.5
            py = y * scale + 0.5
            pz = z * scale + 0.5
            bx = px.astype(jnp.int32)
            by = py.astype(jnp.int32)
            bz = pz.astype(jnp.int32)
            fx = px - bx.astype(jnp.float32)
            fy = py - by.astype(jnp.float32)
            fz = pz - bz.astype(jnp.float32)
            off = _SP_OFF[l] if l < N_SP else l * T
            if direct:
                r2 = res * res
                hx = (bx, jnp.minimum(bx + 1, res - 1))
                sy0 = by * res
                hy = (sy0, jnp.minimum(sy0 + res, (res - 1) * res))
                sz0 = bz * r2
                sz1 = jnp.minimum(sz0 + r2, (res - 1) * r2)
                hz = (sz0 + off, sz1 + off)
                comb = lambda a, b, c: a + b + c
            else:
                hx = (bx, bx + 1)
                hy0 = by * P2I
                hy = (hy0, hy0 + P2I)
                hz0 = bz * P3I
                hz = (hz0, hz0 + P3I)
                comb = lambda a, b, c: ((a ^ b ^ c) & MASK) + off
            wx = (1.0 - fx, fx)
            wy = (1.0 - fy, fy)
            wz = (1.0 - fz, fz)
            wyz = [[wy[j] * wz[k] for k in range(2)] for j in range(2)]
            for c in range(8):
                cx, cy, cz = c & 1, (c >> 1) & 1, (c >> 2) & 1
                idx = comb(hx[cx], hy[cy], hz[cz])
                w = wx[cx] * wyz[cy][cz]
                p0 = g * 128 + c * 16
                ib[pl.ds(p0, 16)] = idx
                wb[pl.ds(p0, 16)] = w
            return 0

        lax.fori_loop(0, GROUPS, body, 0)

    def pass_b_pair(l, wb, rb):
        def body(g, _):
            a0 = jnp.zeros((16,), jnp.float32)
            a1 = jnp.zeros((16,), jnp.float32)
            for c in range(8):
                p0 = g * 128 + c * 16
                wc = wb[pl.ds(p0, 16)]
                ridx = iota + p0
                f0 = plsc.load_gather(rb, [ridx, zidx])
                f1 = plsc.load_gather(rb, [ridx, oidx])
                a0 = a0 + wc * f0
                a1 = a1 + wc * f1
            ep = iota * 32 + (g * 512 + 2 * l)
            plsc.store_scatter(ev, [ep], a0)
            plsc.store_scatter(ev, [ep + 1], a1)
            return 0

        lax.fori_loop(0, GROUPS, body, 0)

    def pass_b_split(l, wb, rb0, rb1):
        def body(g, _):
            a0 = jnp.zeros((16,), jnp.float32)
            a1 = jnp.zeros((16,), jnp.float32)
            for c in range(8):
                p0 = g * 128 + c * 16
                wc = wb[pl.ds(p0, 16)]
                f0 = rb0[pl.ds(p0, 16)]
                f1 = rb1[pl.ds(p0, 16)]
                a0 = a0 + wc * f0
                a1 = a1 + wc * f1
            ep = iota * 32 + (g * 512 + 2 * l)
            plsc.store_scatter(ev, [ep], a0)
            plsc.store_scatter(ev, [ep + 1], a1)
            return 0

        lax.fori_loop(0, GROUPS, body, 0)

    def chunk(k, _):
        base = wid * PW + k * C
        pltpu.sync_copy(x_hbm.at[pl.ds(base, C)], xv)
        pltpu.sync_copy(y_hbm.at[pl.ds(base, C)], yv)
        pltpu.sync_copy(z_hbm.at[pl.ds(base, C)], zv)
        cps = [None] * N_LEVELS
        for l in range(N_LEVELS):
            par = l % 2
            ib, wb = (iv0, wv0) if par == 0 else (iv1, wv1)
            sb = sem0 if par == 0 else sem1
            pass_a(l, ib, wb)
            if l < N_SP:
                rp = rpa if par == 0 else rpb
                cps[l] = (pltpu.async_copy(spt.at[ib], rp, sb),)
            else:
                rb0, rb1 = (r0a, r1a) if par == 0 else (r0b, r1b)
                cps[l] = (pltpu.async_copy(tab0_hbm.at[ib], rb0, sb),
                          pltpu.async_copy(tab1_hbm.at[ib], rb1, sb))
            if l >= 1:
                for cp in cps[l - 1]:
                    cp.wait()
                pp = (l - 1) % 2
                wbp = wv0 if pp == 0 else wv1
                if l - 1 < N_SP:
                    pass_b_pair(l - 1, wbp, rpa if pp == 0 else rpb)
                else:
                    rb0p, rb1p = (r0a, r1a) if pp == 0 else (r0b, r1b)
                    pass_b_split(l - 1, wbp, rb0p, rb1p)
        for cp in cps[N_LEVELS - 1]:
            cp.wait()
        pass_b_split(N_LEVELS - 1, wv1, r0b, r1b)
        pltpu.sync_copy(ev, emb_hbm.at[pl.ds(base * 32, C * 32)])
        return 0

    lax.fori_loop(0, CHUNKS, chunk, 0)


@functools.partial(
    pl.kernel,
    mesh=plsc.VectorSubcoreMesh(core_axis_name="c", subcore_axis_name="s"),
    compiler_params=pltpu.CompilerParams(needs_layout_passes=False),
    out_type=jax.ShapeDtypeStruct((NP * 32,), jnp.float32),
    scratch_types=[
        pltpu.VMEM((C,), jnp.float32),
        pltpu.VMEM((C,), jnp.float32),
        pltpu.VMEM((C,), jnp.float32),
        pltpu.VMEM((C * 8,), jnp.int32),
        pltpu.VMEM((C * 8,), jnp.int32),
        pltpu.VMEM((C * 8,), jnp.float32),
        pltpu.VMEM((C * 8,), jnp.float32),
        pltpu.VMEM((C * 8, 2), jnp.float32),
        pltpu.VMEM((C * 8, 2), jnp.float32),
        pltpu.VMEM((C * 8,), jnp.float32),
        pltpu.VMEM((C * 8,), jnp.float32),
        pltpu.VMEM((C * 8,), jnp.float32),
        pltpu.VMEM((C * 8,), jnp.float32),
        pltpu.VMEM((C * 32,), jnp.float32),
        pltpu.VMEM((_SLAB,), jnp.float32),
        pltpu.VMEM((_SLAB,), jnp.float32),
        pltpu.VMEM((_SLAB, 2), jnp.float32),
        pltpu.VMEM_SHARED((SP_ROWS, 2), jnp.float32),
        pltpu.SemaphoreType.DMA,
        pltpu.SemaphoreType.DMA,
    ],
)
def _sc_encode(x_hbm, y_hbm, z_hbm, tab0_hbm, tab1_hbm, emb_hbm, *rest):
    _sc_body(x_hbm, y_hbm, z_hbm, tab0_hbm, tab1_hbm, emb_hbm, *rest)


def _mlp_body(shs_ref, aux_ref, emb_ref, w1a_ref, w1b_ref, w1c_ref, w2_ref, w3_ref, out_ref):
    s = shs_ref[...]
    ss = jnp.sum(s * s, axis=1, keepdims=True)
    sn = s / jnp.maximum(jnp.sqrt(ss), 1e-12)
    h = (jnp.dot(sn, w1a_ref[...], preferred_element_type=jnp.float32)
         + jnp.dot(aux_ref[...], w1b_ref[...], preferred_element_type=jnp.float32)
         + jnp.dot(emb_ref[...], w1c_ref[...], preferred_element_type=jnp.float32))
    h = jnp.maximum(h, 0.0)
    h = jnp.maximum(jnp.dot(h, w2_ref[...], preferred_element_type=jnp.float32), 0.0)
    out_ref[...] = jnp.dot(h, w3_ref[...], preferred_element_type=jnp.float32)


def _mlp(shs, aux, emb, W1a, W1b, W1c, W2, W3):
    n = shs.shape[0]
    grid = n // BN
    full = lambda shp: pl.BlockSpec(shp, lambda i: (0, 0))
    return pl.pallas_call(
        _mlp_body,
        grid=(grid,),
        in_specs=[
            pl.BlockSpec((BN, 48), lambda i: (i, 0)),
            pl.BlockSpec((BN, 10), lambda i: (i, 0)),
            pl.BlockSpec((BN, N_LEVELS * N_FEATS), lambda i: (i, 0)),
            full((48, WIDTH)), full((10, WIDTH)), full((N_LEVELS * N_FEATS, WIDTH)),
            full((WIDTH, WIDTH)), full((WIDTH, 1)),
        ],
        out_specs=pl.BlockSpec((BN, 1), lambda i: (i, 0)),
        out_shape=jax.ShapeDtypeStruct((n, 1), jnp.float32),
    )(shs, aux, emb, W1a, W1b, W1c, W2, W3)


def kernel(shs, rotations, scales, viewdirs, xyz, table, W1, W2, W3):
    n = shs.shape[0]
    shs2 = shs.reshape(n, -1)
    aux = jnp.concatenate([viewdirs, rotations, scales], axis=1)
    pad = NP - n
    xp = jnp.pad(xyz[:, 0], (0, pad))
    yp = jnp.pad(xyz[:, 1], (0, pad))
    zp = jnp.pad(xyz[:, 2], (0, pad))
    tab = table.reshape(N_LEVELS * T, N_FEATS)
    emb = _sc_encode(xp, yp, zp, tab[:, 0], tab[:, 1]).reshape(NP, 32)[:n]
    W1a, W1b, W1c = W1[:48], W1[48:58], W1[58:]
    return _mlp(shs2, aux, emb, W1a, W1b, W1c, W2, W3)
